# baseline (speedup 1.0000x reference)
import dataclasses
import functools

import jax
import jax.numpy as jnp
from jax import lax
from jax.experimental import pallas as pl
from jax.experimental.pallas import tpu as pltpu
from jax.experimental.pallas import tpu_sc as plsc

BATCH = 16384
HIST = 200
DPAD = 8
NW = 32
PER_W = BATCH // NW
G = 16
NGROUPS = PER_W // G
ROWS = G * HIST

_LOG2E = 1.4426950408889634
_LN2_HI = 0.693145751953125
_LN2_LO = 1.4286067653302226e-06


def _exp_f32(x):
    t = x * _LOG2E
    ni = (t + jnp.where(t >= 0.0, 0.5, -0.5)).astype(jnp.int32)
    ni = jnp.clip(ni, -126, 127)
    nf = ni.astype(jnp.float32)
    r = x - nf * _LN2_HI
    r = r - nf * _LN2_LO
    p = 1.0 / 720.0
    for c in (1.0 / 120.0, 1.0 / 24.0, 1.0 / 6.0, 0.5, 1.0, 1.0):
        p = p * r + c
    scale = lax.bitcast_convert_type((ni + 127) << 23, jnp.float32)
    return p * scale


def _sc_body(w_hbm, lflat_hbm, x_hbm, y_hbm, out_hbm,
             xv, exall, yv, lbuf, ey, outb, sem):
    cid = lax.axis_index("c")
    sid = lax.axis_index("s")
    wid = sid * 2 + cid
    wbase = wid * PER_W

    pltpu.sync_copy(x_hbm.at[pl.ds(wbase, PER_W)], xv)
    pltpu.sync_copy(y_hbm.at[pl.ds(wbase, PER_W)], yv)
    pltpu.async_copy(w_hbm.at[xv], exall, sem).wait()

    iota = lax.iota(jnp.int32, G)
    rowbase = iota * HIST

    @pl.loop(0, NGROUPS)
    def _group(g):
        pltpu.sync_copy(lflat_hbm.at[pl.ds((wbase + g * G) * HIST, ROWS)], lbuf)
        pltpu.async_copy(w_hbm.at[lbuf], ey, sem).wait()

        gxrow = iota + g * G
        exd = [plsc.load_gather(exall, [gxrow, jnp.full((G,), d, jnp.int32)])
               for d in range(5)]
        ygrp = yv[pl.ds(g * G, G)]

        def masked_score(k):
            rowv = rowbase + k
            sc = exd[0] * plsc.load_gather(ey, [rowv, jnp.full((G,), 0, jnp.int32)])
            for d in range(1, 5):
                sc = sc + exd[d] * plsc.load_gather(
                    ey, [rowv, jnp.full((G,), d, jnp.int32)])
            lvals = plsc.load_gather(lbuf, [rowv])
            return jnp.where(lvals > 0, _exp_f32(sc), 0.0)

        UNROLL = 4

        def step(j, carry):
            denom, numer = carry
            k0 = j * UNROLL
            ms = [masked_score(k0 + u) for u in range(UNROLL)]
            denom = denom + ((ms[0] + ms[1]) + (ms[2] + ms[3]))
            for u in range(UNROLL):
                numer = jnp.where(ygrp == k0 + u, ms[u], numer)
            return denom, numer

        zeros = jnp.zeros((G,), jnp.float32)
        denom, numer = lax.fori_loop(0, HIST // UNROLL, step, (zeros, zeros))
        outb[pl.ds(g * G, G)] = numer / denom

    pltpu.sync_copy(outb, out_hbm.at[pl.ds(wbase, PER_W)])


def kernel(x, y, l, W):
    w8 = jnp.pad(W, ((0, 0), (0, DPAD - W.shape[1])))
    w8 = w8.astype(jnp.bfloat16).astype(jnp.float32)
    lflat = l.reshape(-1).astype(jnp.int32)
    mesh = plsc.VectorSubcoreMesh(core_axis_name="c", subcore_axis_name="s")
    cp = pltpu.CompilerParams()
    for fld, val in (("needs_layout_passes", False),
                     ("use_tc_tiling_on_sc", False)):
        if fld in pltpu.CompilerParams.__dataclass_fields__:
            cp = dataclasses.replace(cp, **{fld: val})
    run = pl.kernel(
        _sc_body,
        out_type=jax.ShapeDtypeStruct((BATCH,), jnp.float32),
        mesh=mesh,
        scratch_types=[
            pltpu.VMEM((PER_W,), jnp.int32),
            pltpu.VMEM((PER_W, DPAD), jnp.float32),
            pltpu.VMEM((PER_W,), jnp.int32),
            pltpu.VMEM((ROWS,), jnp.int32),
            pltpu.VMEM((ROWS, DPAD), jnp.float32),
            pltpu.VMEM((PER_W,), jnp.float32),
            pltpu.SemaphoreType.DMA,
        ],
        compiler_params=cp,
    )
    return run(w8, lflat, x.astype(jnp.int32), y.astype(jnp.int32))


# speedup vs baseline: 1.3133x; 1.3133x over previous
import dataclasses
import functools

import jax
import jax.numpy as jnp
from jax import lax
from jax.experimental import pallas as pl
from jax.experimental.pallas import tpu as pltpu
from jax.experimental.pallas import tpu_sc as plsc

BATCH = 16384
HIST = 200
DPAD = 8
NW = 32
PER_W = BATCH // NW
G = 16
NGROUPS = PER_W // G
ROWS = G * HIST

_LOG2E = 1.4426950408889634
_LN2_HI = 0.693145751953125
_LN2_LO = 1.4286067653302226e-06


def _exp_f32(x):
    t = x * _LOG2E
    ni = (t + jnp.where(t >= 0.0, 0.5, -0.5)).astype(jnp.int32)
    ni = jnp.clip(ni, -126, 127)
    nf = ni.astype(jnp.float32)
    r = x - nf * _LN2_HI
    r = r - nf * _LN2_LO
    p = 1.0 / 720.0
    for c in (1.0 / 120.0, 1.0 / 24.0, 1.0 / 6.0, 0.5, 1.0, 1.0):
        p = p * r + c
    scale = lax.bitcast_convert_type((ni + 127) << 23, jnp.float32)
    return p * scale


NPAIR = NGROUPS // 2


def _sc_body(w_hbm, lflat_hbm, x_hbm, y_hbm, out_hbm,
             xv, exall, yv, lbuf0, lbuf1, ey0, ey1, outb, sem0, sem1):
    cid = lax.axis_index("c")
    sid = lax.axis_index("s")
    wid = sid * 2 + cid
    wbase = wid * PER_W

    pltpu.sync_copy(x_hbm.at[pl.ds(wbase, PER_W)], xv)
    pltpu.sync_copy(y_hbm.at[pl.ds(wbase, PER_W)], yv)
    pltpu.async_copy(w_hbm.at[xv], exall, sem0).wait()

    iota = lax.iota(jnp.int32, G)
    rowbase = iota * HIST

    def issue(g, lbuf, ey, sem):
        pltpu.sync_copy(lflat_hbm.at[pl.ds((wbase + g * G) * HIST, ROWS)], lbuf)
        pltpu.make_async_copy(w_hbm.at[lbuf], ey, sem).start()

    def compute(g, lbuf, ey):
        gxrow = iota + g * G
        exd = [plsc.load_gather(exall, [gxrow, jnp.full((G,), d, jnp.int32)])
               for d in range(5)]
        ygrp = yv[pl.ds(g * G, G)]

        def masked_score(k):
            rowv = rowbase + k
            sc = exd[0] * plsc.load_gather(ey, [rowv, jnp.full((G,), 0, jnp.int32)])
            for d in range(1, 5):
                sc = sc + exd[d] * plsc.load_gather(
                    ey, [rowv, jnp.full((G,), d, jnp.int32)])
            lvals = plsc.load_gather(lbuf, [rowv])
            return jnp.where(lvals > 0, _exp_f32(sc), 0.0)

        UNROLL = 4

        def step(j, carry):
            denom, numer = carry
            k0 = j * UNROLL
            ms = [masked_score(k0 + u) for u in range(UNROLL)]
            denom = denom + ((ms[0] + ms[1]) + (ms[2] + ms[3]))
            for u in range(UNROLL):
                numer = jnp.where(ygrp == k0 + u, ms[u], numer)
            return denom, numer

        zeros = jnp.zeros((G,), jnp.float32)
        denom, numer = lax.fori_loop(0, HIST // UNROLL, step, (zeros, zeros))
        outb[pl.ds(g * G, G)] = numer / denom

    issue(0, lbuf0, ey0, sem0)

    @pl.loop(0, NPAIR)
    def _pair(p):
        g0 = 2 * p
        issue(g0 + 1, lbuf1, ey1, sem1)
        pltpu.make_async_copy(w_hbm.at[lbuf0], ey0, sem0).wait()
        compute(g0, lbuf0, ey0)

        @pl.when(p < NPAIR - 1)
        def _():
            issue(g0 + 2, lbuf0, ey0, sem0)

        pltpu.make_async_copy(w_hbm.at[lbuf1], ey1, sem1).wait()
        compute(g0 + 1, lbuf1, ey1)

    pltpu.sync_copy(outb, out_hbm.at[pl.ds(wbase, PER_W)])


def kernel(x, y, l, W):
    w8 = jnp.pad(W, ((0, 0), (0, DPAD - W.shape[1])))
    w8 = w8.astype(jnp.bfloat16).astype(jnp.float32)
    lflat = l.reshape(-1).astype(jnp.int32)
    mesh = plsc.VectorSubcoreMesh(core_axis_name="c", subcore_axis_name="s")
    cp = pltpu.CompilerParams()
    for fld, val in (("needs_layout_passes", False),
                     ("use_tc_tiling_on_sc", False)):
        if fld in pltpu.CompilerParams.__dataclass_fields__:
            cp = dataclasses.replace(cp, **{fld: val})
    run = pl.kernel(
        _sc_body,
        out_type=jax.ShapeDtypeStruct((BATCH,), jnp.float32),
        mesh=mesh,
        scratch_types=[
            pltpu.VMEM((PER_W,), jnp.int32),
            pltpu.VMEM((PER_W, DPAD), jnp.float32),
            pltpu.VMEM((PER_W,), jnp.int32),
            pltpu.VMEM((ROWS,), jnp.int32),
            pltpu.VMEM((ROWS,), jnp.int32),
            pltpu.VMEM((ROWS, DPAD), jnp.float32),
            pltpu.VMEM((ROWS, DPAD), jnp.float32),
            pltpu.VMEM((PER_W,), jnp.float32),
            pltpu.SemaphoreType.DMA,
            pltpu.SemaphoreType.DMA,
        ],
        compiler_params=cp,
    )
    return run(w8, lflat, x.astype(jnp.int32), y.astype(jnp.int32))


# speedup vs baseline: 1.3468x; 1.0255x over previous
import dataclasses
import functools

import jax
import jax.numpy as jnp
from jax import lax
from jax.experimental import pallas as pl
from jax.experimental.pallas import tpu as pltpu
from jax.experimental.pallas import tpu_sc as plsc

BATCH = 16384
HIST = 200
DPAD = 8
NW = 32
PER_W = BATCH // NW
G = 16
NGROUPS = PER_W // G
ROWS = G * HIST

_LOG2E = 1.4426950408889634
_LN2_HI = 0.693145751953125
_LN2_LO = 1.4286067653302226e-06


def _exp_f32(x):
    t = x * _LOG2E
    ni = (t + jnp.where(t >= 0.0, 0.5, -0.5)).astype(jnp.int32)
    ni = jnp.clip(ni, -126, 127)
    nf = ni.astype(jnp.float32)
    r = x - nf * _LN2_HI
    r = r - nf * _LN2_LO
    p = 1.0 / 120.0
    for c in (1.0 / 24.0, 1.0 / 6.0, 0.5, 1.0, 1.0):
        p = p * r + c
    scale = lax.bitcast_convert_type((ni + 127) << 23, jnp.float32)
    return p * scale


NPAIR = NGROUPS // 2


def _sc_body(w_hbm, lflat_hbm, x_hbm, y_hbm, out_hbm,
             xv, exall, yv, lbuf0, lbuf1, ey0, ey1, outb, sem0, sem1):
    cid = lax.axis_index("c")
    sid = lax.axis_index("s")
    wid = sid * 2 + cid
    wbase = wid * PER_W

    pltpu.sync_copy(x_hbm.at[pl.ds(wbase, PER_W)], xv)
    pltpu.sync_copy(y_hbm.at[pl.ds(wbase, PER_W)], yv)
    pltpu.async_copy(w_hbm.at[xv], exall, sem0).wait()

    iota = lax.iota(jnp.int32, G)
    rowbase = iota * HIST

    def issue(g, lbuf, ey, sem):
        pltpu.sync_copy(lflat_hbm.at[pl.ds((wbase + g * G) * HIST, ROWS)], lbuf)
        pltpu.make_async_copy(w_hbm.at[lbuf], ey, sem).start()

    def compute(g, lbuf, ey):
        gxrow = iota + g * G
        exd = [plsc.load_gather(exall, [gxrow, jnp.full((G,), d, jnp.int32)])
               for d in range(5)]

        def masked_score(rowv):
            sc = exd[0] * plsc.load_gather(ey, [rowv, jnp.full((G,), 0, jnp.int32)])
            for d in range(1, 5):
                sc = sc + exd[d] * plsc.load_gather(
                    ey, [rowv, jnp.full((G,), d, jnp.int32)])
            lvals = plsc.load_gather(lbuf, [rowv])
            return jnp.where(lvals > 0, _exp_f32(sc), 0.0)

        UNROLL = 8

        def step(j, denom):
            k0 = j * UNROLL
            ms = [masked_score(rowbase + (k0 + u)) for u in range(UNROLL)]
            while len(ms) > 1:
                ms = [a + b for a, b in zip(ms[::2], ms[1::2])]
            return denom + ms[0]

        zeros = jnp.zeros((G,), jnp.float32)
        denom = lax.fori_loop(0, HIST // UNROLL, step, zeros)
        numer = masked_score(rowbase + yv[pl.ds(g * G, G)])
        outb[pl.ds(g * G, G)] = numer / denom

    issue(0, lbuf0, ey0, sem0)

    @pl.loop(0, NPAIR)
    def _pair(p):
        g0 = 2 * p
        issue(g0 + 1, lbuf1, ey1, sem1)
        pltpu.make_async_copy(w_hbm.at[lbuf0], ey0, sem0).wait()
        compute(g0, lbuf0, ey0)

        @pl.when(p < NPAIR - 1)
        def _():
            issue(g0 + 2, lbuf0, ey0, sem0)

        pltpu.make_async_copy(w_hbm.at[lbuf1], ey1, sem1).wait()
        compute(g0 + 1, lbuf1, ey1)

    pltpu.sync_copy(outb, out_hbm.at[pl.ds(wbase, PER_W)])


def kernel(x, y, l, W):
    w8 = jnp.pad(W, ((0, 0), (0, DPAD - W.shape[1])))
    w8 = w8.astype(jnp.bfloat16).astype(jnp.float32)
    lflat = l.reshape(-1).astype(jnp.int32)
    mesh = plsc.VectorSubcoreMesh(core_axis_name="c", subcore_axis_name="s")
    cp = pltpu.CompilerParams()
    for fld, val in (("needs_layout_passes", False),
                     ("use_tc_tiling_on_sc", False)):
        if fld in pltpu.CompilerParams.__dataclass_fields__:
            cp = dataclasses.replace(cp, **{fld: val})
    run = pl.kernel(
        _sc_body,
        out_type=jax.ShapeDtypeStruct((BATCH,), jnp.float32),
        mesh=mesh,
        scratch_types=[
            pltpu.VMEM((PER_W,), jnp.int32),
            pltpu.VMEM((PER_W, DPAD), jnp.float32),
            pltpu.VMEM((PER_W,), jnp.int32),
            pltpu.VMEM((ROWS,), jnp.int32),
            pltpu.VMEM((ROWS,), jnp.int32),
            pltpu.VMEM((ROWS, DPAD), jnp.float32),
            pltpu.VMEM((ROWS, DPAD), jnp.float32),
            pltpu.VMEM((PER_W,), jnp.float32),
            pltpu.SemaphoreType.DMA,
            pltpu.SemaphoreType.DMA,
        ],
        compiler_params=cp,
    )
    return run(w8, lflat, x.astype(jnp.int32), y.astype(jnp.int32))


# speedup vs baseline: 1.6004x; 1.1883x over previous
import dataclasses
import functools

import jax
import jax.numpy as jnp
from jax import lax
from jax.experimental import pallas as pl
from jax.experimental.pallas import tpu as pltpu
from jax.experimental.pallas import tpu_sc as plsc

BATCH = 16384
HIST = 200
D = 5
NW = 32
PER_W = BATCH // NW
G = 16
NGROUPS = PER_W // G
ROWS = G * HIST

_LOG2E = 1.4426950408889634
_LN2_HI = 0.693145751953125
_LN2_LO = 1.4286067653302226e-06


def _exp_f32(x):
    t = x * _LOG2E
    ni = (t + jnp.where(t >= 0.0, 0.5, -0.5)).astype(jnp.int32)
    ni = jnp.clip(ni, -126, 127)
    nf = ni.astype(jnp.float32)
    r = x - nf * _LN2_HI
    r = r - nf * _LN2_LO
    p = 1.0 / 120.0
    for c in (1.0 / 24.0, 1.0 / 6.0, 0.5, 1.0, 1.0):
        p = p * r + c
    scale = lax.bitcast_convert_type((ni + 127) << 23, jnp.float32)
    return p * scale


NPAIR = NGROUPS // 2


def _sc_body(w_hbm, lflat_hbm, x_hbm, y_hbm, out_hbm,
             xv, exall, yv, lbuf0, lbuf1, ey0, ey1, outb, sem0, sem1):
    cid = lax.axis_index("c")
    sid = lax.axis_index("s")
    wid = sid * 2 + cid
    wbase = wid * PER_W

    pltpu.sync_copy(x_hbm.at[pl.ds(wbase, PER_W)], xv)
    pltpu.sync_copy(y_hbm.at[pl.ds(wbase, PER_W)], yv)
    pltpu.async_copy(w_hbm.at[xv], exall, sem0).wait()

    iota = lax.iota(jnp.int32, G)
    rowbase = iota * HIST

    def issue(g, lbuf, ey, sem):
        pltpu.sync_copy(lflat_hbm.at[pl.ds((wbase + g * G) * HIST, ROWS)], lbuf)
        pltpu.make_async_copy(w_hbm.at[lbuf], ey, sem).start()

    def compute(g, lbuf, ey):
        gxrow = iota + g * G
        exd = [plsc.load_gather(exall, [gxrow, jnp.full((G,), d, jnp.int32)])
               for d in range(D)]

        def score_at(rowv):
            sc = exd[0] * plsc.load_gather(ey, [rowv, jnp.full((G,), 0, jnp.int32)])
            for d in range(1, D):
                sc = sc + exd[d] * plsc.load_gather(
                    ey, [rowv, jnp.full((G,), d, jnp.int32)])
            return sc

        UNROLL = 8

        def step(j, denom):
            k0 = j * UNROLL
            ms = []
            for u in range(UNROLL):
                k = k0 + u
                sc = score_at(iota + k * G)
                lvals = lbuf[pl.ds(k * G, G)]
                ms.append(jnp.where(lvals > 0, _exp_f32(sc), 0.0))
            while len(ms) > 1:
                ms = [a + b for a, b in zip(ms[::2], ms[1::2])]
            return denom + ms[0]

        zeros = jnp.zeros((G,), jnp.float32)
        denom = lax.fori_loop(0, HIST // UNROLL, step, zeros)
        ygrp = yv[pl.ds(g * G, G)]
        rowy = iota + ygrp * G
        ly = plsc.load_gather(lbuf, [rowy])
        numer = jnp.where(ly > 0, _exp_f32(score_at(rowy)), 0.0)
        outb[pl.ds(g * G, G)] = numer / denom

    issue(0, lbuf0, ey0, sem0)

    @pl.loop(0, NPAIR)
    def _pair(p):
        g0 = 2 * p
        issue(g0 + 1, lbuf1, ey1, sem1)
        pltpu.make_async_copy(w_hbm.at[lbuf0], ey0, sem0).wait()
        compute(g0, lbuf0, ey0)

        @pl.when(p < NPAIR - 1)
        def _():
            issue(g0 + 2, lbuf0, ey0, sem0)

        pltpu.make_async_copy(w_hbm.at[lbuf1], ey1, sem1).wait()
        compute(g0 + 1, lbuf1, ey1)

    pltpu.sync_copy(outb, out_hbm.at[pl.ds(wbase, PER_W)])


def kernel(x, y, l, W):
    w5 = W.astype(jnp.bfloat16).astype(jnp.float32)
    lflat = (l.astype(jnp.int32)
             .reshape(BATCH // G, G, HIST)
             .transpose(0, 2, 1)
             .reshape(-1))
    mesh = plsc.VectorSubcoreMesh(core_axis_name="c", subcore_axis_name="s")
    cp = pltpu.CompilerParams()
    for fld, val in (("needs_layout_passes", False),
                     ("use_tc_tiling_on_sc", False)):
        if fld in pltpu.CompilerParams.__dataclass_fields__:
            cp = dataclasses.replace(cp, **{fld: val})
    run = pl.kernel(
        _sc_body,
        out_type=jax.ShapeDtypeStruct((BATCH,), jnp.float32),
        mesh=mesh,
        scratch_types=[
            pltpu.VMEM((PER_W,), jnp.int32),
            pltpu.VMEM((PER_W, D), jnp.float32),
            pltpu.VMEM((PER_W,), jnp.int32),
            pltpu.VMEM((ROWS,), jnp.int32),
            pltpu.VMEM((ROWS,), jnp.int32),
            pltpu.VMEM((ROWS, D), jnp.float32),
            pltpu.VMEM((ROWS, D), jnp.float32),
            pltpu.VMEM((PER_W,), jnp.float32),
            pltpu.SemaphoreType.DMA,
            pltpu.SemaphoreType.DMA,
        ],
        compiler_params=cp,
    )
    return run(w5, lflat, x.astype(jnp.int32), y.astype(jnp.int32))


# speedup vs baseline: 1.7238x; 1.0771x over previous
import dataclasses
import functools

import jax
import jax.numpy as jnp
from jax import lax
from jax.experimental import pallas as pl
from jax.experimental.pallas import tpu as pltpu
from jax.experimental.pallas import tpu_sc as plsc

BATCH = 16384
HIST = 200
D = 5
NW = 32
PER_W = BATCH // NW
G = 16
NGROUPS = PER_W // G
ROWS = G * HIST

_LOG2E = 1.4426950408889634
_LN2_HI = 0.693145751953125
_LN2_LO = 1.4286067653302226e-06


def _exp_f32(x):
    t = x * _LOG2E
    ni = (t + jnp.where(t >= 0.0, 0.5, -0.5)).astype(jnp.int32)
    ni = jnp.clip(ni, -126, 127)
    nf = ni.astype(jnp.float32)
    r = x - nf * _LN2_HI
    r = r - nf * _LN2_LO
    p = 1.0 / 120.0
    for c in (1.0 / 24.0, 1.0 / 6.0, 0.5, 1.0, 1.0):
        p = p * r + c
    scale = lax.bitcast_convert_type((ni + 127) << 23, jnp.float32)
    return p * scale


NPAIR = NGROUPS // 2
PADROWS = ROWS + G
NKB = (HIST + G - 1) // G


def _perm(x, idx):
    dnums = lax.GatherDimensionNumbers(
        offset_dims=(), collapsed_slice_dims=(0,), start_index_map=(0,))
    return lax.gather(x, idx[:, None], dnums, (1,),
                      mode=lax.GatherScatterMode.PROMISE_IN_BOUNDS)


def _sc_body(w_hbm, lflat_hbm, x_hbm, y_hbm, out_hbm,
             xv, exall, yv, lbuf0, lbuf1, ey0, ey1, outb, sem0, sem1):
    cid = lax.axis_index("c")
    sid = lax.axis_index("s")
    wid = sid * 2 + cid
    wbase = wid * PER_W

    pltpu.sync_copy(x_hbm.at[pl.ds(wbase, PER_W)], xv)
    pltpu.sync_copy(y_hbm.at[pl.ds(wbase, PER_W)], yv)
    pltpu.async_copy(w_hbm.at[xv], exall, sem0).wait()

    iota = lax.iota(jnp.int32, G)
    for lb in (lbuf0, lbuf1):
        lb[pl.ds(ROWS, G)] = jnp.zeros((G,), jnp.int32)
    perms = [iota ^ s for s in (8, 4, 2, 1)]

    def issue(g, lbuf, ey, sem):
        pltpu.sync_copy(lflat_hbm.at[pl.ds((wbase + g * G) * HIST, ROWS)],
                        lbuf.at[pl.ds(0, ROWS)])
        pltpu.make_async_copy(w_hbm.at[lbuf], ey, sem).start()

    def compute(g, lbuf, ey):
        def body(i_loc, outv):
            i = g * G + i_loc
            isplat = jnp.full((G,), 0, jnp.int32) + i
            exd = [plsc.load_gather(exall, [isplat, jnp.full((G,), d, jnp.int32)])
                   for d in range(D)]
            y_i = plsc.load_gather(yv, [isplat])
            rbase = i_loc * HIST
            denacc = jnp.zeros((G,), jnp.float32)
            numacc = jnp.zeros((G,), jnp.float32)
            for kb in range(NKB):
                k0 = kb * G
                rowv = iota + (rbase + k0)
                sc = exd[0] * plsc.load_gather(
                    ey, [rowv, jnp.full((G,), 0, jnp.int32)])
                for d in range(1, D):
                    sc = sc + exd[d] * plsc.load_gather(
                        ey, [rowv, jnp.full((G,), d, jnp.int32)])
                lvals = lbuf[pl.ds(rbase + k0, G)]
                mask = lvals > 0
                if kb == NKB - 1:
                    mask = mask & (iota + k0 < HIST)
                masked = jnp.where(mask, _exp_f32(sc), 0.0)
                denacc = denacc + masked
                numacc = numacc + jnp.where(iota + k0 == y_i, masked, 0.0)
            for p in perms:
                denacc = denacc + _perm(denacc, p)
                numacc = numacc + _perm(numacc, p)
            return jnp.where(iota == i_loc, numacc / denacc, outv)

        outv = lax.fori_loop(0, G, body, jnp.zeros((G,), jnp.float32))
        outb[pl.ds(g * G, G)] = outv

    issue(0, lbuf0, ey0, sem0)

    @pl.loop(0, NPAIR)
    def _pair(p):
        g0 = 2 * p
        issue(g0 + 1, lbuf1, ey1, sem1)
        pltpu.make_async_copy(w_hbm.at[lbuf0], ey0, sem0).wait()
        compute(g0, lbuf0, ey0)

        @pl.when(p < NPAIR - 1)
        def _():
            issue(g0 + 2, lbuf0, ey0, sem0)

        pltpu.make_async_copy(w_hbm.at[lbuf1], ey1, sem1).wait()
        compute(g0 + 1, lbuf1, ey1)

    pltpu.sync_copy(outb, out_hbm.at[pl.ds(wbase, PER_W)])


def kernel(x, y, l, W):
    w5 = W.astype(jnp.bfloat16).astype(jnp.float32)
    lflat = l.reshape(-1)
    mesh = plsc.VectorSubcoreMesh(core_axis_name="c", subcore_axis_name="s")
    cp = pltpu.CompilerParams()
    for fld, val in (("needs_layout_passes", False),
                     ("use_tc_tiling_on_sc", False)):
        if fld in pltpu.CompilerParams.__dataclass_fields__:
            cp = dataclasses.replace(cp, **{fld: val})
    run = pl.kernel(
        _sc_body,
        out_type=jax.ShapeDtypeStruct((BATCH,), jnp.float32),
        mesh=mesh,
        scratch_types=[
            pltpu.VMEM((PER_W,), jnp.int32),
            pltpu.VMEM((PER_W, D), jnp.float32),
            pltpu.VMEM((PER_W,), jnp.int32),
            pltpu.VMEM((PADROWS,), jnp.int32),
            pltpu.VMEM((PADROWS,), jnp.int32),
            pltpu.VMEM((PADROWS, D), jnp.float32),
            pltpu.VMEM((PADROWS, D), jnp.float32),
            pltpu.VMEM((PER_W,), jnp.float32),
            pltpu.SemaphoreType.DMA,
            pltpu.SemaphoreType.DMA,
        ],
        compiler_params=cp,
    )
    return run(w5, lflat, x.astype(jnp.int32), y.astype(jnp.int32))
